# baseline (speedup 1.0000x reference)
import math

import jax
import jax.numpy as jnp
from jax import lax
from jax.experimental import pallas as pl
from jax.experimental.pallas import tpu as pltpu
from jax.experimental.pallas import tpu_sc as plsc

D_MODEL_K = 64
VOCAB_K = 1000000
SCALE_K = math.sqrt(float(D_MODEL_K))

NUM_CORES = 2
NUM_SUBCORES = 16
NUM_WORKERS = NUM_CORES * NUM_SUBCORES
LANES = 16

CHUNK = 128


def _emb_kernel(idx_hbm, table_hbm, out_hbm, idx_v, rows_v, sem):
    chunks_per_worker = idx_v.shape[0]
    wid = lax.axis_index("c") * NUM_SUBCORES + lax.axis_index("s")
    chunk_base = wid * chunks_per_worker

    pltpu.sync_copy(idx_hbm.at[pl.ds(chunk_base, chunks_per_worker)], idx_v)

    @pl.loop(0, chunks_per_worker)
    def _(j):
        pltpu.async_copy(table_hbm.at[idx_v.at[j]], rows_v, sem).wait()

        @pl.loop(0, CHUNK)
        def _(r):
            for c in range(D_MODEL_K // LANES):
                slc = (pl.ds(r, 1), pl.ds(c * LANES, LANES))
                rows_v.at[slc][...] = rows_v.at[slc][...] * SCALE_K

        row_base = (chunk_base + j) * CHUNK
        pltpu.sync_copy(rows_v, out_hbm.at[pl.ds(row_base, CHUNK)])


def kernel(x, table):
    batch, seq = x.shape
    total = batch * seq
    d = table.shape[1]
    assert total % (NUM_WORKERS * CHUNK) == 0
    chunks_per_worker = total // (NUM_WORKERS * CHUNK)

    idx = x.reshape(total // CHUNK, CHUNK)

    mesh = plsc.VectorSubcoreMesh(core_axis_name="c", subcore_axis_name="s")
    emb = pl.kernel(
        _emb_kernel,
        out_type=jax.ShapeDtypeStruct((total, d), table.dtype),
        mesh=mesh,
        compiler_params=pltpu.CompilerParams(use_tc_tiling_on_sc=False),
        scratch_types=[
            pltpu.VMEM((chunks_per_worker, CHUNK), jnp.int32),
            pltpu.VMEM((CHUNK, d), jnp.float32),
            pltpu.SemaphoreType.DMA,
        ],
    )
    out = emb(idx, table)
    return out.reshape(batch, seq, d)


# speedup vs baseline: 1.2062x; 1.2062x over previous
import math

import jax
import jax.numpy as jnp
from jax import lax
from jax.experimental import pallas as pl
from jax.experimental.pallas import tpu as pltpu
from jax.experimental.pallas import tpu_sc as plsc

D_MODEL_K = 64
SCALE_K = math.sqrt(float(D_MODEL_K))

NUM_CORES = 2
NUM_SUBCORES = 16
NUM_WORKERS = NUM_CORES * NUM_SUBCORES
LANES = 16

CHUNK = 128
NBUF = 4


def _emb_kernel(idx_hbm, table_hbm, out_hbm, idx_v, rows_v, gsem, wsem):
    chunks = idx_v.shape[0]
    wid = lax.axis_index("c") * NUM_SUBCORES + lax.axis_index("s")
    chunk_base = wid * chunks

    def fire_gather(b, j):
        pltpu.async_copy(table_hbm.at[idx_v.at[j]], rows_v.at[b], gsem.at[b])

    def wait_gather(b, j):
        pltpu.make_async_copy(
            table_hbm.at[idx_v.at[j]], rows_v.at[b], gsem.at[b]
        ).wait()

    def fire_write(b, j):
        dst = out_hbm.at[pl.ds((chunk_base + j) * CHUNK, CHUNK)]
        pltpu.async_copy(rows_v.at[b], dst, wsem.at[b])

    def wait_write(b, j):
        dst = out_hbm.at[pl.ds((chunk_base + j) * CHUNK, CHUNK)]
        pltpu.make_async_copy(rows_v.at[b], dst, wsem.at[b]).wait()

    def scale(b):
        buf = rows_v.at[b]

        @pl.loop(0, CHUNK, unroll=8)
        def _(r):
            for c in range(D_MODEL_K // LANES):
                slc = (pl.ds(r, 1), pl.ds(c * LANES, LANES))
                buf.at[slc][...] = buf.at[slc][...] * SCALE_K

    pltpu.sync_copy(idx_hbm.at[pl.ds(chunk_base, chunks)], idx_v)

    for b in range(NBUF - 1):
        fire_gather(b, b)

    @pl.loop(0, chunks // NBUF - 1)
    def _(g):
        for b in range(NBUF):
            j = g * NBUF + b
            pb = (b + NBUF - 1) % NBUF
            wait_gather(b, j)
            scale(b)
            fire_write(b, j)
            if b == 0:
                @pl.when(g > 0)
                def _():
                    wait_write(pb, j - 1)
            else:
                wait_write(pb, j - 1)
            fire_gather(pb, j + NBUF - 1)

    j0 = chunks - NBUF
    wait_gather(0, j0)
    scale(0)
    fire_write(0, j0)
    wait_write(NBUF - 1, j0 - 1)
    fire_gather(NBUF - 1, chunks - 1)
    for b in range(1, NBUF):
        j = j0 + b
        wait_gather(b, j)
        scale(b)
        fire_write(b, j)

    for b in range(NBUF):
        wait_write(b, j0 + b)


def kernel(x, table):
    batch, seq = x.shape
    total = batch * seq
    d = table.shape[1]
    assert total % (NUM_WORKERS * CHUNK * NBUF) == 0
    chunks_per_worker = total // (NUM_WORKERS * CHUNK)

    idx = x.reshape(total // CHUNK, CHUNK)

    mesh = plsc.VectorSubcoreMesh(core_axis_name="c", subcore_axis_name="s")
    emb = pl.kernel(
        _emb_kernel,
        out_type=jax.ShapeDtypeStruct((total, d), table.dtype),
        mesh=mesh,
        compiler_params=pltpu.CompilerParams(use_tc_tiling_on_sc=False),
        scratch_types=[
            pltpu.VMEM((chunks_per_worker, CHUNK), jnp.int32),
            pltpu.VMEM((NBUF, CHUNK, d), jnp.float32),
            pltpu.SemaphoreType.DMA((NBUF,)),
            pltpu.SemaphoreType.DMA((NBUF,)),
        ],
    )
    out = emb(idx, table)
    return out.reshape(batch, seq, d)
